# baseline (speedup 1.0000x reference)
import functools

import jax
import jax.numpy as jnp
from jax import lax
from jax.experimental import pallas as pl
from jax.experimental.pallas import tpu as pltpu

_EPS = 2.220446049250313e-16


def _moe_lora_body(x_ref, aw_ref, bf_ref, out_ref, *, A, E, R):
    x = x_ref[...]
    Bt = x.shape[0]
    ER = E * R
    NH = A * ER
    NW = aw_ref.shape[1]
    hc = jnp.dot(x, aw_ref[...], preferred_element_type=jnp.float32)
    col = lax.broadcasted_iota(jnp.int32, (Bt, NW), 1)
    is_logit = (col >= NH) & (col < NH + E)
    m = jnp.max(jnp.where(is_logit, hc, -jnp.inf), axis=1, keepdims=True)
    e_idx = jnp.min(jnp.where(is_logit & (hc == m), col - NH, E),
                    axis=1, keepdims=True)
    col_e = jnp.where(col < NH, (col // R) % E, E)
    g = jnp.where(col_e == e_idx, hc, 0.0)
    for a in range(A):
        out = jnp.dot(g, bf_ref[a], preferred_element_type=jnp.float32)
        ex = jnp.exp(out).astype(jnp.bfloat16).astype(jnp.float32)
        out_ref[a, :, :] = jnp.log(jnp.where(ex == 0.0, _EPS, ex))


def kernel(x, w_gate, lora_a, lora_b):
    B, C = x.shape
    A, E, R, _ = lora_a.shape
    ER = E * R
    NH = A * ER
    NW = NH + E + 1
    a_flat = lora_a.transpose(3, 0, 1, 2).reshape(C, NH)
    aw = jnp.concatenate(
        [a_flat, w_gate, jnp.zeros((C, NW - NH - E), jnp.float32)], axis=1)
    b_flat = jnp.zeros((A, NW, C), jnp.float32)
    bt = lora_b.transpose(0, 1, 3, 2).reshape(A, ER, C)
    for a in range(A):
        b_flat = b_flat.at[a, a * ER:(a + 1) * ER].set(bt[a])
    Bt = 1024
    return pl.pallas_call(
        functools.partial(_moe_lora_body, A=A, E=E, R=R),
        grid=(B // Bt,),
        in_specs=[
            pl.BlockSpec((Bt, C), lambda i: (i, 0)),
            pl.BlockSpec((C, NW), lambda i: (0, 0)),
            pl.BlockSpec((A, NW, C), lambda i: (0, 0, 0)),
        ],
        out_specs=pl.BlockSpec((A, Bt, C), lambda i: (0, i, 0)),
        out_shape=jax.ShapeDtypeStruct((A, B, C), jnp.float32),
        compiler_params=pltpu.CompilerParams(
            dimension_semantics=("arbitrary",),
        ),
    )(x, aw, b_flat)
